# baseline (speedup 1.0000x reference)
import functools

import jax
import jax.numpy as jnp
from jax import lax
from jax.experimental import pallas as pl
from jax.experimental.pallas import tpu as pltpu
from jax.experimental.pallas import tpu_sc as plsc

_EMBED = 64
_NMETA = 16
_CTX = 128
_P = 50
_WIN = 128
_NW = 32
_BBLK = 256


def _sc_gather_combine(emb_table, flat_ids, meta):
    n = flat_ids.shape[1]
    bsz, p = meta.shape[0], meta.shape[1]
    nwin = n // _WIN
    wpw = nwin // _NW
    wpp = bsz // _WIN
    mesh = plsc.VectorSubcoreMesh(core_axis_name="c", subcore_axis_name="s")

    nbuf = 5
    assert wpw % nbuf == 0

    @functools.partial(
        pl.kernel,
        out_type=jax.ShapeDtypeStruct((n, _CTX), jnp.float32),
        mesh=mesh,
        compiler_params=pltpu.CompilerParams(use_tc_tiling_on_sc=False),
        scratch_types=[
            pltpu.VMEM((wpw * _WIN,), jnp.int32),
            [pltpu.VMEM((_WIN, _EMBED), jnp.float32) for _ in range(nbuf)],
            pltpu.VMEM((_WIN, _CTX - 80), jnp.float32),
            pltpu.SemaphoreType.DMA,
            [pltpu.SemaphoreType.DMA for _ in range(nbuf)],
            [pltpu.SemaphoreType.DMA for _ in range(nbuf)],
        ],
    )
    def k(tbl_hbm, idx_hbm, meta_hbm, out_hbm, idx_v, rows, zeros_v,
          sidx, sg, so):
        wid = lax.axis_index("s") * 2 + lax.axis_index("c")
        base_win = wid * wpw
        pltpu.async_copy(
            idx_hbm.at[0, pl.ds(base_win * _WIN, wpw * _WIN)], idx_v, sidx
        ).wait()
        zeros = jnp.zeros((16,), jnp.float32)

        @pl.loop(0, _WIN)
        def _(r):
            for c in range(0, _CTX - 80, 16):
                zeros_v[r, pl.ds(c, 16)] = zeros

        def start_gather(t, j):
            return pltpu.async_copy(
                tbl_hbm.at[idx_v.at[pl.ds(t * _WIN, _WIN)]], rows[j], sg[j])

        def start_outs(t, j):
            w = base_win + t
            p_ = w // wpp
            b0 = (w % wpp) * _WIN
            r0 = w * _WIN
            pltpu.async_copy(
                rows[j], out_hbm.at[pl.ds(r0, _WIN), pl.ds(0, _EMBED)], so[j])
            pltpu.async_copy(
                meta_hbm.at[pl.ds(b0, _WIN), p_, :],
                out_hbm.at[pl.ds(r0, _WIN), pl.ds(_EMBED, _NMETA)], so[j])
            pltpu.async_copy(
                zeros_v, out_hbm.at[pl.ds(r0, _WIN), pl.ds(80, _CTX - 80)],
                so[j])

        def wait_outs(j):
            pltpu.make_async_copy(
                rows[j], out_hbm.at[pl.ds(0, _WIN), pl.ds(0, _EMBED)],
                so[j]).wait()
            pltpu.make_async_copy(
                meta_hbm.at[pl.ds(0, _WIN), 0, :],
                out_hbm.at[pl.ds(0, _WIN), pl.ds(_EMBED, _NMETA)],
                so[j]).wait()
            pltpu.make_async_copy(
                zeros_v, out_hbm.at[pl.ds(0, _WIN), pl.ds(80, _CTX - 80)],
                so[j]).wait()

        for j in range(nbuf):
            start_gather(j, j)
        for j in range(nbuf):
            pltpu.make_async_copy(
                tbl_hbm.at[idx_v.at[pl.ds(0, _WIN)]], rows[j], sg[j]).wait()
            start_outs(j, j)

        @pl.loop(nbuf, wpw, step=nbuf)
        def _(t):
            for j in range(nbuf):
                wait_outs(j)
                start_gather(t + j, j)
            for j in range(nbuf):
                pltpu.make_async_copy(
                    tbl_hbm.at[idx_v.at[pl.ds(0, _WIN)]], rows[j],
                    sg[j]).wait()
                start_outs(t + j, j)

        for j in range(nbuf):
            wait_outs(j)

    return k(emb_table, flat_ids, meta)


def _tc_mlp2_body(comb_ref, w1p_ref, b1_ref, w2_ref, b2_ref,
                  gamma_ref, beta_ref, out_ref):
    p, nb = comb_ref.shape[0], comb_ref.shape[1]
    x = comb_ref[...].reshape(p * nb, _CTX)
    h = jnp.dot(x, w1p_ref[...], preferred_element_type=jnp.float32) + b1_ref[...]
    h = jnp.maximum(h, 0.0)
    h = jnp.dot(h, w2_ref[...], preferred_element_type=jnp.float32) + b2_ref[...]
    mean = jnp.mean(h, axis=-1, keepdims=True)
    d = h - mean
    var = jnp.mean(d * d, axis=-1, keepdims=True)
    y = d * jax.lax.rsqrt(var + 1e-5) * gamma_ref[...] + beta_ref[...]
    out_ref[...] = jnp.max(y.reshape(p, nb, _CTX), axis=0)


def _tc_mlp2(comb3, w1p, b1, w2, b2, gamma, beta):
    p, bsz = comb3.shape[0], comb3.shape[1]
    fixed = lambda i: (0, 0)
    return pl.pallas_call(
        _tc_mlp2_body,
        grid=(bsz // _BBLK,),
        in_specs=[
            pl.BlockSpec((p, _BBLK, _CTX), lambda i: (0, i, 0)),
            pl.BlockSpec((_CTX, _CTX), fixed),
            pl.BlockSpec((1, _CTX), fixed),
            pl.BlockSpec((_CTX, _CTX), fixed),
            pl.BlockSpec((1, _CTX), fixed),
            pl.BlockSpec((1, _CTX), fixed),
            pl.BlockSpec((1, _CTX), fixed),
        ],
        out_specs=pl.BlockSpec((_BBLK, _CTX), lambda i: (i, 0)),
        out_shape=jax.ShapeDtypeStruct((bsz, _CTX), jnp.float32),
    )(comb3, w1p, b1, w2, b2, gamma, beta)


def kernel(pattern_ids, pattern_metadata, emb_table, W1, b1, W2, b2, gamma, beta):
    bsz, p = pattern_ids.shape
    n = bsz * p
    flat_ids = pattern_ids.T.reshape(1, n).astype(jnp.int32)
    comb = _sc_gather_combine(emb_table, flat_ids, pattern_metadata)
    comb3 = comb.reshape(p, bsz, _CTX)
    w1p = jnp.concatenate([W1, jnp.zeros((_CTX - W1.shape[0], _CTX), W1.dtype)], axis=0)
    return _tc_mlp2(
        comb3, w1p, b1.reshape(1, _CTX), W2, b2.reshape(1, _CTX),
        gamma.reshape(1, _CTX), beta.reshape(1, _CTX),
    )


# speedup vs baseline: 3.1744x; 3.1744x over previous
import functools

import jax
import jax.numpy as jnp
from jax.experimental import pallas as pl
from jax.experimental.pallas import tpu as pltpu
from jax.experimental.pallas import tpu_sc as plsc

_EMBED = 64
_NMETA = 16
_CTX = 128
_P = 50

_GATHER_WINDOW = 128
_BBLK = 256


def _sc_gather(emb_table, idx2d):
    nwin = idx2d.shape[0]
    n = nwin * _GATHER_WINDOW
    mesh = plsc.VectorSubcoreMesh(core_axis_name="c", subcore_axis_name="s")

    @functools.partial(
        pl.kernel,
        out_type=jax.ShapeDtypeStruct((n, _EMBED), emb_table.dtype),
        mesh=mesh,
        compiler_params=pltpu.CompilerParams(use_tc_tiling_on_sc=False),
    )
    def gather_kernel(tbl_hbm, idx_hbm, out_hbm):
        def body(i_vmem, o_vmem):
            pltpu.sync_copy(tbl_hbm.at[i_vmem.at[0]], o_vmem)

        pltpu.emit_pipeline(
            body,
            grid=(nwin,),
            in_specs=[
                pl.BlockSpec((1, _GATHER_WINDOW), index_map=lambda i: (i, 0))
            ],
            out_specs=[
                pl.BlockSpec((_GATHER_WINDOW, _EMBED), index_map=lambda i: (i, 0))
            ],
            core_axis_name=("c", "s"),
            dimension_semantics=(pltpu.PARALLEL,),
        )(idx_hbm, out_hbm)

    return gather_kernel(emb_table, idx2d)


def _tc_mlp_body(emb_ref, meta_ref, w1a_ref, w1b_ref, b1_ref, w2_ref, b2_ref,
                 gamma_ref, beta_ref, out_ref):
    p, nb = emb_ref.shape[0], emb_ref.shape[1]
    bf = jnp.bfloat16
    emb = emb_ref[...].reshape(p * nb, _EMBED).astype(bf)
    meta = meta_ref[...].reshape(p * nb, _NMETA).astype(bf)
    h = (
        jnp.dot(emb, w1a_ref[...].astype(bf), preferred_element_type=jnp.float32)
        + jnp.dot(meta, w1b_ref[...].astype(bf), preferred_element_type=jnp.float32)
        + b1_ref[...]
    )
    h = jnp.maximum(h, 0.0)
    h = jnp.dot(h.astype(bf), w2_ref[...].astype(bf),
                preferred_element_type=jnp.float32) + b2_ref[...]
    mean = jnp.mean(h, axis=-1, keepdims=True)
    d = h - mean
    var = jnp.mean(d * d, axis=-1, keepdims=True)
    y = d * jax.lax.rsqrt(var + 1e-5) * gamma_ref[...] + beta_ref[...]
    out_ref[...] = jnp.max(y.reshape(p, nb, _CTX), axis=0)


def _tc_mlp(embeds3, meta3, w1a, w1b, b1, w2, b2, gamma, beta):
    p, bsz = embeds3.shape[0], embeds3.shape[1]
    fixed = lambda i: (0, 0)
    return pl.pallas_call(
        _tc_mlp_body,
        grid=(bsz // _BBLK,),
        in_specs=[
            pl.BlockSpec((p, _BBLK, _EMBED), lambda i: (0, i, 0)),
            pl.BlockSpec((p, _BBLK, _NMETA), lambda i: (0, i, 0)),
            pl.BlockSpec((_EMBED, _CTX), fixed),
            pl.BlockSpec((_NMETA, _CTX), fixed),
            pl.BlockSpec((1, _CTX), fixed),
            pl.BlockSpec((_CTX, _CTX), fixed),
            pl.BlockSpec((1, _CTX), fixed),
            pl.BlockSpec((1, _CTX), fixed),
            pl.BlockSpec((1, _CTX), fixed),
        ],
        out_specs=pl.BlockSpec((_BBLK, _CTX), lambda i: (i, 0)),
        out_shape=jax.ShapeDtypeStruct((bsz, _CTX), jnp.float32),
    )(embeds3, meta3, w1a, w1b, b1, w2, b2, gamma, beta)


def kernel(pattern_ids, pattern_metadata, emb_table, W1, b1, W2, b2, gamma, beta):
    bsz, p = pattern_ids.shape
    n = bsz * p
    idx2d = pattern_ids.T.reshape(n // _GATHER_WINDOW, _GATHER_WINDOW).astype(jnp.int32)
    embeds = _sc_gather(emb_table, idx2d)
    embeds3 = embeds.reshape(p, bsz, _EMBED)
    meta3 = pattern_metadata.transpose(1, 0, 2)
    w1a = W1[:_EMBED]
    w1b = W1[_EMBED:]
    out = _tc_mlp(
        embeds3, meta3, w1a, w1b,
        b1.reshape(1, _CTX), W2, b2.reshape(1, _CTX),
        gamma.reshape(1, _CTX), beta.reshape(1, _CTX),
    )
    return out


# speedup vs baseline: 4.1083x; 1.2942x over previous
import functools

import jax
import jax.numpy as jnp
from jax.experimental import pallas as pl
from jax.experimental.pallas import tpu as pltpu
from jax.experimental.pallas import tpu_sc as plsc

_EMBED = 64
_NMETA = 16
_CTX = 128
_WIN = 128
_MBLK = 32


def _sc_gather(emb_table, idx2d):
    nwin = idx2d.shape[0]
    n = nwin * _WIN
    mesh = plsc.VectorSubcoreMesh(core_axis_name="c", subcore_axis_name="s")

    @functools.partial(
        pl.kernel,
        out_type=jax.ShapeDtypeStruct((n, _CTX), emb_table.dtype),
        mesh=mesh,
        compiler_params=pltpu.CompilerParams(use_tc_tiling_on_sc=False),
    )
    def gather_kernel(tbl_hbm, idx_hbm, out_hbm):
        def body(i_vmem, o_vmem):
            pltpu.sync_copy(tbl_hbm.at[i_vmem.at[0]], o_vmem)

        pltpu.emit_pipeline(
            body,
            grid=(nwin,),
            in_specs=[
                pl.BlockSpec((1, _WIN), index_map=lambda i: (i, 0))
            ],
            out_specs=[
                pl.BlockSpec((_WIN, _EMBED), index_map=lambda i: (i, 0))
            ],
            core_axis_name=("c", "s"),
            dimension_semantics=(pltpu.PARALLEL,),
        )(idx_hbm, out_hbm)

    return gather_kernel(emb_table, idx2d)


def _tc_body(emb_ref, m8_ref, w1al_ref, w1b8_ref, b1_ref, w2_ref, b2_ref,
             gamma_ref, beta_ref, out_ref):
    p, ns, nm = emb_ref.shape[0], emb_ref.shape[1], emb_ref.shape[2]
    bf = jnp.bfloat16
    rows = p * nm
    lane = jax.lax.broadcasted_iota(jnp.int32, (rows, _CTX), 1)
    m8 = m8_ref[...].reshape(rows, _CTX).astype(bf)
    w1al = w1al_ref[...].astype(bf)
    w2 = w2_ref[...].astype(bf)
    for j in range(ns):
        x = emb_ref[:, j].reshape(rows, _CTX)
        x = jnp.where(lane < _EMBED, x, 0.0).astype(bf)
        h = (
            jnp.dot(x, w1al, preferred_element_type=jnp.float32)
            + jnp.dot(m8, w1b8_ref[j].astype(bf),
                      preferred_element_type=jnp.float32)
            + b1_ref[...]
        )
        h = jnp.maximum(h, 0.0)
        h = jnp.dot(h.astype(bf), w2,
                    preferred_element_type=jnp.float32) + b2_ref[...]
        mean = jnp.mean(h, axis=-1, keepdims=True)
        d = h - mean
        var = jnp.mean(d * d, axis=-1, keepdims=True)
        y = d * jax.lax.rsqrt(var + 1e-5) * gamma_ref[...] + beta_ref[...]
        out_ref[j] = jnp.max(y.reshape(p, nm, _CTX), axis=0)


def _tc_mlp(emb4, m83, w1al, w1b8, b1, w2, b2, gamma, beta):
    p, ns, nm = emb4.shape[0], emb4.shape[1], emb4.shape[2]
    fixed = lambda i: (0, 0)
    return pl.pallas_call(
        _tc_body,
        grid=(nm // _MBLK,),
        in_specs=[
            pl.BlockSpec((p, ns, _MBLK, _CTX), lambda i: (0, 0, i, 0)),
            pl.BlockSpec((p, _MBLK, _CTX), lambda i: (0, i, 0)),
            pl.BlockSpec((_CTX, _CTX), fixed),
            pl.BlockSpec((ns, _CTX, _CTX), lambda i: (0, 0, 0)),
            pl.BlockSpec((1, _CTX), fixed),
            pl.BlockSpec((_CTX, _CTX), fixed),
            pl.BlockSpec((1, _CTX), fixed),
            pl.BlockSpec((1, _CTX), fixed),
            pl.BlockSpec((1, _CTX), fixed),
        ],
        out_specs=pl.BlockSpec((ns, _MBLK, _CTX), lambda i: (0, i, 0)),
        out_shape=jax.ShapeDtypeStruct((ns, nm, _CTX), jnp.float32),
    )(emb4, m83, w1al, w1b8, b1, w2, b2, gamma, beta)


def kernel(pattern_ids, pattern_metadata, emb_table, W1, b1, W2, b2, gamma, beta):
    bsz, p = pattern_ids.shape
    n = bsz * p
    nm = bsz // 8
    ids_g = pattern_ids.T.reshape(p, nm, 8).transpose(0, 2, 1)
    idx2d = ids_g.reshape(n // _WIN, _WIN).astype(jnp.int32)
    embeds = _sc_gather(emb_table, idx2d)
    emb4 = embeds.reshape(p, 8, nm, _CTX)
    m83 = pattern_metadata.transpose(1, 0, 2).reshape(p, nm, 8 * _NMETA)
    w1a = W1[:_EMBED]
    w1b = W1[_EMBED:]
    w1al = jnp.concatenate([w1a, jnp.zeros((_CTX - _EMBED, _CTX), W1.dtype)], axis=0)
    w1b8 = jnp.zeros((8, _CTX, _CTX), W1.dtype)
    for j in range(8):
        w1b8 = w1b8.at[j, j * _NMETA:(j + 1) * _NMETA, :].set(w1b)
    pooled = _tc_mlp(
        emb4, m83, w1al, w1b8,
        b1.reshape(1, _CTX), W2, b2.reshape(1, _CTX),
        gamma.reshape(1, _CTX), beta.reshape(1, _CTX),
    )
    return pooled.transpose(1, 0, 2).reshape(bsz, _CTX)


# speedup vs baseline: 4.2415x; 1.0324x over previous
import functools

import jax
import jax.numpy as jnp
from jax.experimental import pallas as pl
from jax.experimental.pallas import tpu as pltpu
from jax.experimental.pallas import tpu_sc as plsc

_EMBED = 64
_NMETA = 16
_CTX = 128
_WIN = 128
_MBLK = 64


def _sc_gather(emb_table, idx2d):
    nwin = idx2d.shape[0]
    n = nwin * _WIN
    mesh = plsc.VectorSubcoreMesh(core_axis_name="c", subcore_axis_name="s")

    @functools.partial(
        pl.kernel,
        out_type=jax.ShapeDtypeStruct((n, _CTX), emb_table.dtype),
        mesh=mesh,
        compiler_params=pltpu.CompilerParams(use_tc_tiling_on_sc=False),
    )
    def gather_kernel(tbl_hbm, idx_hbm, out_hbm):
        def body(i_vmem, o_vmem):
            pltpu.sync_copy(tbl_hbm.at[i_vmem.at[0]], o_vmem)

        pltpu.emit_pipeline(
            body,
            grid=(nwin,),
            in_specs=[
                pl.BlockSpec((1, _WIN), index_map=lambda i: (i, 0))
            ],
            out_specs=[
                pl.BlockSpec((_WIN, _EMBED), index_map=lambda i: (i, 0))
            ],
            core_axis_name=("c", "s"),
            dimension_semantics=(pltpu.PARALLEL,),
        )(idx_hbm, out_hbm)

    return gather_kernel(emb_table, idx2d)


def _tc_body(emb_ref, m8_ref, w1al_ref, w1b8_ref, b1_ref, w2_ref, b2_ref,
             gamma_ref, beta_ref, out_ref):
    p, ns, nm = emb_ref.shape[0], emb_ref.shape[1], emb_ref.shape[2]
    bf = jnp.bfloat16
    rows = p * nm
    lane = jax.lax.broadcasted_iota(jnp.int32, (rows, _CTX), 1)
    m8 = m8_ref[...].reshape(rows, _CTX).astype(bf)
    w1al = w1al_ref[...].astype(bf)
    w2 = w2_ref[...].astype(bf)
    for j in range(ns):
        x = emb_ref[:, j].reshape(rows, _CTX)
        x = jnp.where(lane < _EMBED, x, 0.0).astype(bf)
        h = (
            jnp.dot(x, w1al, preferred_element_type=jnp.float32)
            + jnp.dot(m8, w1b8_ref[j].astype(bf),
                      preferred_element_type=jnp.float32)
            + b1_ref[...]
        )
        h = jnp.maximum(h, 0.0)
        h = jnp.dot(h.astype(bf), w2,
                    preferred_element_type=jnp.float32) + b2_ref[...]
        mean = jnp.mean(h, axis=-1, keepdims=True)
        d = h - mean
        var = jnp.mean(d * d, axis=-1, keepdims=True)
        y = d * jax.lax.rsqrt(var + 1e-5) * gamma_ref[...] + beta_ref[...]
        out_ref[j] = jnp.max(y.reshape(p, nm, _CTX), axis=0)


def _tc_mlp(emb4, m83, w1al, w1b8, b1, w2, b2, gamma, beta):
    p, ns, nm = emb4.shape[0], emb4.shape[1], emb4.shape[2]
    fixed = lambda i: (0, 0)
    return pl.pallas_call(
        _tc_body,
        grid=(nm // _MBLK,),
        in_specs=[
            pl.BlockSpec((p, ns, _MBLK, _CTX), lambda i: (0, 0, i, 0)),
            pl.BlockSpec((p, _MBLK, _CTX), lambda i: (0, i, 0)),
            pl.BlockSpec((_CTX, _CTX), fixed),
            pl.BlockSpec((ns, _CTX, _CTX), lambda i: (0, 0, 0)),
            pl.BlockSpec((1, _CTX), fixed),
            pl.BlockSpec((_CTX, _CTX), fixed),
            pl.BlockSpec((1, _CTX), fixed),
            pl.BlockSpec((1, _CTX), fixed),
            pl.BlockSpec((1, _CTX), fixed),
        ],
        out_specs=pl.BlockSpec((ns, _MBLK, _CTX), lambda i: (0, i, 0)),
        out_shape=jax.ShapeDtypeStruct((ns, nm, _CTX), jnp.float32),
    )(emb4, m83, w1al, w1b8, b1, w2, b2, gamma, beta)


def kernel(pattern_ids, pattern_metadata, emb_table, W1, b1, W2, b2, gamma, beta):
    bsz, p = pattern_ids.shape
    n = bsz * p
    nm = bsz // 8
    ids_g = pattern_ids.T.reshape(p, nm, 8).transpose(0, 2, 1)
    idx2d = ids_g.reshape(n // _WIN, _WIN).astype(jnp.int32)
    embeds = _sc_gather(emb_table, idx2d)
    emb4 = embeds.reshape(p, 8, nm, _CTX)
    m83 = pattern_metadata.transpose(1, 0, 2).reshape(p, nm, 8 * _NMETA)
    w1a = W1[:_EMBED]
    w1b = W1[_EMBED:]
    w1al = jnp.concatenate([w1a, jnp.zeros((_CTX - _EMBED, _CTX), W1.dtype)], axis=0)
    w1b8 = jnp.zeros((8, _CTX, _CTX), W1.dtype)
    for j in range(8):
        w1b8 = w1b8.at[j, j * _NMETA:(j + 1) * _NMETA, :].set(w1b)
    pooled = _tc_mlp(
        emb4, m83, w1al, w1b8,
        b1.reshape(1, _CTX), W2, b2.reshape(1, _CTX),
        gamma.reshape(1, _CTX), beta.reshape(1, _CTX),
    )
    return pooled.transpose(1, 0, 2).reshape(bsz, _CTX)
